# Initial kernel scaffold; baseline (speedup 1.0000x reference)
import jax
import jax.numpy as jnp
from jax.experimental import pallas as pl


def kernel(token_ids, embedding_matrix):
    raise NotImplementedError("write your pallas kernel here")


# speedup vs baseline: 1.5734x; 1.5734x over previous
import functools

import jax
import jax.numpy as jnp
from jax import lax
from jax.experimental import pallas as pl
from jax.experimental.pallas import tpu as pltpu
from jax.experimental.pallas import tpu_sc as plsc

VOCAB = 1_000_000
D_MODEL = 64

NC = 2
NS = 16
NW = NC * NS

CHUNK = 128


def _sc_gather(idx_flat, table, b_total):
    b_per_w = b_total // NW
    n_chunks = b_per_w // CHUNK
    mesh = plsc.VectorSubcoreMesh(core_axis_name="c", subcore_axis_name="s")

    @functools.partial(
        pl.kernel,
        out_type=jax.ShapeDtypeStruct((b_total, D_MODEL), jnp.float32),
        mesh=mesh,
        scratch_types=[
            pltpu.VMEM((2, CHUNK), jnp.int32),
            pltpu.VMEM((2, CHUNK, D_MODEL), jnp.float32),
            pltpu.SemaphoreType.DMA,
            pltpu.SemaphoreType.DMA,
        ],
        compiler_params=pltpu.CompilerParams(use_tc_tiling_on_sc=False),
    )
    def k(idx_hbm, table_hbm, out_hbm, idx_v, rows_v, gsem, osem):
        wid = lax.axis_index("s") * NC + lax.axis_index("c")
        base = wid * b_per_w

        def body(g, _):
            buf = lax.rem(g, 2)
            off = base + g * CHUNK
            pltpu.sync_copy(idx_hbm.at[pl.ds(off, CHUNK)], idx_v.at[buf])
            pltpu.async_copy(table_hbm.at[idx_v.at[buf]], rows_v.at[buf], gsem).wait()
            pltpu.sync_copy(rows_v.at[buf], out_hbm.at[pl.ds(off, CHUNK)])
            return ()

        lax.fori_loop(0, n_chunks, body, (), unroll=False)

    return k(idx_flat, table)


def kernel(token_ids, embedding_matrix):
    n, s = token_ids.shape
    b_total = n * s
    idx_flat = token_ids.reshape(b_total).astype(jnp.int32)
    out = _sc_gather(idx_flat, embedding_matrix, b_total)
    return out.reshape(n, s, D_MODEL)


# speedup vs baseline: 1.8770x; 1.1930x over previous
import functools

import jax
import jax.numpy as jnp
from jax import lax
from jax.experimental import pallas as pl
from jax.experimental.pallas import tpu as pltpu
from jax.experimental.pallas import tpu_sc as plsc

VOCAB = 1_000_000
D_MODEL = 64

NC = 2
NS = 16
NW = NC * NS

CHUNK = 128
GH = 4
NBUF = 2 * GH


def _sc_gather(idx2d, table, b_total):
    n_chunks = b_total // CHUNK
    cpw = n_chunks // NW
    ng = cpw // GH
    mesh = plsc.VectorSubcoreMesh(core_axis_name="c", subcore_axis_name="s")

    @functools.partial(
        pl.kernel,
        out_type=jax.ShapeDtypeStruct((b_total, D_MODEL), jnp.float32),
        mesh=mesh,
        scratch_types=[
            pltpu.VMEM((cpw, CHUNK), jnp.int32),
            pltpu.VMEM((NBUF, CHUNK, D_MODEL), jnp.float32),
        ]
        + [pltpu.SemaphoreType.DMA] * (2 * NBUF),
        compiler_params=pltpu.CompilerParams(use_tc_tiling_on_sc=False),
    )
    def k(idx_hbm, table_hbm, out_hbm, idx_v, rows_v, *sems):
        gsems, osems = sems[:NBUF], sems[NBUF:]
        wid = lax.axis_index("s") * NC + lax.axis_index("c")
        cbase = wid * cpw

        def gath(cl, b):
            return pltpu.make_async_copy(
                table_hbm.at[idx_v.at[cl]], rows_v.at[b], gsems[b])

        def stor(cl, b):
            off = (cbase + cl) * CHUNK
            return pltpu.make_async_copy(
                rows_v.at[b], out_hbm.at[pl.ds(off, CHUNK)], osems[b])

        pltpu.sync_copy(idx_hbm.at[pl.ds(cbase, cpw)], idx_v)

        for j in range(GH):
            gath(j, j).start()

        def super_body(si, _):
            for p in range(2):
                gi = 2 * si + p
                for j in range(GH):
                    b = (1 - p) * GH + j

                    @pl.when(gi >= 1)
                    def _():
                        stor((gi - 1) * GH + j, b).wait()

                    @pl.when(gi + 1 < ng)
                    def _():
                        gath((gi + 1) * GH + j, b).start()
                for j in range(GH):
                    b = p * GH + j
                    cl = gi * GH + j
                    gath(cl, b).wait()
                    stor(cl, b).start()
            return ()

        lax.fori_loop(0, ng // 2, super_body, (), unroll=False)

        for j in range(GH):
            b = ((ng - 1) % 2) * GH + j
            stor((ng - 1) * GH + j, b).wait()

    return k(idx2d, table)


def kernel(token_ids, embedding_matrix):
    n, s = token_ids.shape
    b_total = n * s
    idx2d = token_ids.reshape(b_total // CHUNK, CHUNK).astype(jnp.int32)
    out = _sc_gather(idx2d, embedding_matrix, b_total)
    return out.reshape(n, s, D_MODEL)
